# baseline (speedup 1.0000x reference)
import functools

import jax
import jax.numpy as jnp
from jax import lax
from jax.experimental import pallas as pl
from jax.experimental.pallas import tpu as pltpu
from jax.experimental.pallas import tpu_sc as plsc

EMBED_DIM = 32
VOCAB1 = 100001
BATCH = 16384
NUM_CORES = 2
NUM_SUBCORES = 16
LANES = 16
DIMS_PER_SC = EMBED_DIM // NUM_CORES
B_PER_TILE = BATCH // NUM_SUBCORES
CHUNK = 128
GCHUNK = 64
NGCHUNK = B_PER_TILE // GCHUNK
FULL_TILES = VOCAB1 // CHUNK
PACKED_ROWS = (FULL_TILES + 1) * LANES
TAIL_START = FULL_TILES * CHUNK

_mesh = plsc.VectorSubcoreMesh(core_axis_name="c", subcore_axis_name="s")


@functools.partial(
    pl.kernel,
    mesh=_mesh,
    out_type=jax.ShapeDtypeStruct((EMBED_DIM, BATCH), jnp.float32),
    scratch_types=[
        pltpu.VMEM_SHARED((PACKED_ROWS, 8 * LANES), jnp.float32),
        pltpu.VMEM((DIMS_PER_SC, CHUNK), jnp.float32),
        pltpu.VMEM((DIMS_PER_SC, CHUNK), jnp.float32),
        pltpu.VMEM((GCHUNK, 8 * LANES), jnp.float32),
        pltpu.VMEM((DIMS_PER_SC, B_PER_TILE), jnp.float32),
        pltpu.VMEM((B_PER_TILE,), jnp.int32),
        pltpu.VMEM((B_PER_TILE,), jnp.int32),
        pltpu.SemaphoreType.DMA,
    ],
    compiler_params=pltpu.CompilerParams(needs_layout_passes=False),
)
def _gather_kernel(idx_hbm, table_t_hbm, tail_t_hbm, out_hbm,
                   shared, chunk_in, rowbuf, gbuf, stage, idxb, rvb, sem):
    c = lax.axis_index("c")
    s = lax.axis_index("s")
    d0 = c * DIMS_PER_SC
    jb = s * B_PER_TILE

    lane_iota = lax.iota(jnp.int32, LANES)
    iota_div8 = lane_iota // 8
    iota_mod8_x16 = (lane_iota % 8) * LANES

    pltpu.sync_copy(idx_hbm.at[pl.ds(jb, B_PER_TILE)], idxb)

    def transpose_chunk_to_rowbuf():
        for d in range(DIMS_PER_SC):
            for g in range(CHUNK // LANES):
                vals = chunk_in[d, pl.ds(g * LANES, LANES)]
                plsc.store_scatter(
                    rowbuf,
                    [iota_div8 + (2 * g), iota_mod8_x16 + d],
                    vals,
                )

    def fill(kk, _):
        k = s + NUM_SUBCORES * kk

        @pl.when(k < FULL_TILES)
        def _():
            pltpu.sync_copy(
                table_t_hbm.at[pl.ds(d0, DIMS_PER_SC), pl.ds(k * CHUNK, CHUNK)],
                chunk_in,
            )
            transpose_chunk_to_rowbuf()
            pltpu.sync_copy(rowbuf, shared.at[pl.ds(k * LANES, LANES), :])

        return _

    lax.fori_loop(0, (FULL_TILES + NUM_SUBCORES - 1) // NUM_SUBCORES, fill, None)

    @pl.when(s == NUM_SUBCORES - 1)
    def _():
        pltpu.sync_copy(tail_t_hbm.at[pl.ds(d0, DIMS_PER_SC), :], chunk_in)
        transpose_chunk_to_rowbuf()
        pltpu.sync_copy(
            rowbuf, shared.at[pl.ds(FULL_TILES * LANES, LANES), :]
        )

    def rows(g, _):
        p0 = g * LANES
        v = idxb[pl.ds(p0, LANES)]
        rvb[pl.ds(p0, LANES)] = v >> 3
        return _

    lax.fori_loop(0, B_PER_TILE // LANES, rows, None)

    plsc.subcore_barrier()

    def chunk(cc, _):
        pltpu.async_copy(
            shared.at[rvb.at[pl.ds(cc * GCHUNK, GCHUNK)]], gbuf, sem
        ).wait()
        p_base = cc * GCHUNK
        for g in range(GCHUNK // LANES):
            vvec = idxb[pl.ds(p_base + g * LANES, LANES)]
            lane_base = (vvec & 7) * LANES
            for d in range(DIMS_PER_SC):
                vals = plsc.load_gather(
                    gbuf, [lane_iota + (g * LANES), lane_base + d]
                )
                stage[d, pl.ds(p_base + g * LANES, LANES)] = vals
        return _

    lax.fori_loop(0, NGCHUNK, chunk, None)

    pltpu.sync_copy(
        stage, out_hbm.at[pl.ds(d0, DIMS_PER_SC), pl.ds(jb, B_PER_TILE)]
    )


def kernel(indices, table):
    tail_t = jnp.pad(table[TAIL_START:, :], ((0, CHUNK - (VOCAB1 - TAIL_START)), (0, 0))).T
    out_t = _gather_kernel(indices.astype(jnp.int32), table.T, tail_t)
    return out_t.T
